# baseline (speedup 1.0000x reference)
import jax
import jax.numpy as jnp
from jax.experimental import pallas as pl


def _copy_body(pos_ref, out_ref):
    out_ref[...] = pos_ref[...]


def kernel(x, pos_table):
    del x
    return pl.pallas_call(
        _copy_body,
        out_shape=jax.ShapeDtypeStruct(pos_table.shape, pos_table.dtype),
    )(pos_table)
